# baseline (speedup 1.0000x reference)
import jax
import jax.numpy as jnp
from jax import lax
from jax.experimental import pallas as pl
from jax.experimental.pallas import tpu as pltpu

_IN = (64, 96, 512)
_C0, _C1 = 128, 300
_W = _C1 - _C0
_R = _IN[0] * _IN[1]
_N = _R * _W
_K = int(0.1 * _N)
_EPS_LOG = 0.001
_I32_MIN = -(2 ** 31)

_BLK = 512
_G = _R // _BLK


def _s32(val):
    val &= 0xFFFFFFFF
    return val - (1 << 32) if val >= (1 << 31) else val


def _body(x_ref, o_ref, vh_buf, vl_buf, c_ref):
    g = pl.program_id(0)

    @pl.when(g == 0)
    def _():
        for t in range(16):
            c_ref[t] = 0

    xs = x_ref[:, _C0:_C1]
    bits = lax.bitcast_convert_type(xs, jnp.int32)
    v = bits ^ (lax.shift_right_arithmetic(bits, 31) & jnp.int32(0x7FFFFFFF))
    vh = lax.shift_right_arithmetic(v, 16)
    vl = (v & 0xFFFF) - 32768
    rows = pl.ds(g * _BLK, _BLK)
    vh_buf[rows, :] = vh.astype(jnp.int16)
    vl_buf[rows, :] = vl.astype(jnp.int16)

    for t in range(1, 16):
        ts = jnp.int32(_s32((t << 28) ^ (1 << 31)))
        c_ref[t] = c_ref[t] + jnp.sum((v < ts).astype(jnp.int32))

    @pl.when(g == _G - 1)
    def _():
        vhb = vh_buf[...]

        lo = jnp.int32(0)
        for t in range(1, 16):
            lo = jnp.where(c_ref[t] <= _K, jnp.int32(_s32(t << 28)), lo)

        def step_hi(i, lo):
            mid = lo | lax.shift_left(jnp.int32(1), 31 - i)
            mh = (lax.shift_right_arithmetic(mid ^ jnp.int32(_I32_MIN), 16)
                  ).astype(jnp.int16)
            c = jnp.sum((vhb < mh).astype(jnp.int32))
            return jnp.where(c <= _K, mid, lo)

        lo = lax.fori_loop(4, 16, step_hi, lo)

        hh = (lax.shift_right_arithmetic(lo ^ jnp.int32(_I32_MIN), 16)
              ).astype(jnp.int16)
        cbase = jnp.sum((vhb < hh).astype(jnp.int32))
        w = jnp.where(vhb == hh, vl_buf[...], jnp.int16(32767))

        def step_lo(i, lo):
            mid = lo | lax.shift_left(jnp.int32(1), 31 - i)
            ml = ((mid & 0xFFFF) - 32768).astype(jnp.int16)
            c = cbase + jnp.sum((w < ml).astype(jnp.int32))
            return jnp.where(c <= _K, mid, lo)

        lo = lax.fori_loop(16, 32, step_lo, lo)
        vk = lo ^ jnp.int32(_I32_MIN)
        fb = vk ^ (lax.shift_right_arithmetic(vk, 31) & jnp.int32(0x7FFFFFFF))
        eps = lax.bitcast_convert_type(fb, jnp.float32)

        vr = (lax.shift_left(vhb.astype(jnp.int32), 16)
              | ((vl_buf[...].astype(jnp.int32) + 32768) & 0xFFFF))
        fbs = vr ^ (lax.shift_right_arithmetic(vr, 31) & jnp.int32(0x7FFFFFFF))
        xsr = lax.bitcast_convert_type(fbs, jnp.float32)
        m = jnp.maximum(eps, jnp.float32(_EPS_LOG))
        xmax = jnp.max(xsr)
        ylo = jnp.log10(m)
        yhi = jnp.log10(jnp.maximum(xmax, m))
        o_ref[...] = (jnp.log10(jnp.maximum(xsr, m)) - ylo) / (yhi - ylo)


def kernel(x):
    x2 = x.reshape(_R, _IN[2])
    out = pl.pallas_call(
        _body,
        grid=(_G,),
        in_specs=[pl.BlockSpec((_BLK, _IN[2]), lambda g: (g, 0))],
        out_specs=pl.BlockSpec((_R, _W), lambda g: (0, 0)),
        out_shape=jax.ShapeDtypeStruct((_R, _W), jnp.float32),
        scratch_shapes=[
            pltpu.VMEM((_R, _W), jnp.int16),
            pltpu.VMEM((_R, _W), jnp.int16),
            pltpu.SMEM((16,), jnp.int32),
        ],
    )(x2)
    return out.reshape(_IN[0], _IN[1], _W)
